# Initial kernel scaffold; baseline (speedup 1.0000x reference)
import jax
import jax.numpy as jnp
from jax.experimental import pallas as pl


def kernel(x, edge_index, batch, W1, b1, W2, b2, W3, b3, Wl, bl):
    raise NotImplementedError("write your pallas kernel here")


# speedup vs baseline: 10.3632x; 10.3632x over previous
import functools

import jax
import jax.numpy as jnp
from jax import lax
from jax.experimental import pallas as pl
from jax.experimental.pallas import tpu as pltpu
from jax.experimental.pallas import tpu_sc as plsc

_N = 10000
_E = 320000
_D = 128
_C = 19
_G = 64

_NC = 2
_NS = 16
_NW = _NC * _NS
_NP = 10240
_STR = _NP // _NS
_K = 80
_EPT = _E // _NW
_CH = _EPT // _K

_BR = 1024
_NB = _NP // _BR

_mesh = plsc.VectorSubcoreMesh(core_axis_name="c", subcore_axis_name="s",
                               num_cores=_NC, num_subcores=_NS)


def _deg_count_body(dst_hbm, ones_hbm, zrows_hbm, out_hbm,
                    acc, idx_v, rows_v):
    c = lax.axis_index("c")
    s = lax.axis_index("s")

    pltpu.sync_copy(zrows_hbm, rows_v)

    @pl.loop(0, _STR // _K)
    def _zero(j):
        pltpu.sync_copy(rows_v, acc.at[pl.ds(s * _STR + j * _K, _K)])

    plsc.subcore_barrier()

    pltpu.sync_copy(ones_hbm, rows_v)
    base = (c * _NS + s) * _EPT

    @pl.loop(0, _CH)
    def _count(i):
        pltpu.sync_copy(dst_hbm.at[pl.ds(base + i * _K, _K)], idx_v)
        pltpu.sync_copy(rows_v, acc.at[idx_v], add=True)

    plsc.subcore_barrier()

    @pl.loop(0, _STR // _K)
    def _out(j):
        row = s * _STR + j * _K
        pltpu.sync_copy(acc.at[pl.ds(row, _K)], rows_v)
        pltpu.sync_copy(rows_v, out_hbm.at[pl.ds(c * _NP + row, _K)])


def _edge_scatter_body(t_hbm, src_hbm, dst_hbm, zrows_hbm, out_hbm,
                       acc, src_v, dst_v, rows_v, sem):
    c = lax.axis_index("c")
    s = lax.axis_index("s")

    @pl.loop(0, _STR // _K)
    def _zero(j):
        pltpu.sync_copy(zrows_hbm.at[pl.ds(0, _K)], rows_v)
        pltpu.sync_copy(rows_v, acc.at[pl.ds(s * _STR + j * _K, _K)])

    plsc.subcore_barrier()

    base = (c * _NS + s) * _EPT

    @pl.loop(0, _CH)
    def _edges(i):
        pltpu.sync_copy(src_hbm.at[pl.ds(base + i * _K, _K)], src_v)
        pltpu.sync_copy(dst_hbm.at[pl.ds(base + i * _K, _K)], dst_v)
        pltpu.async_copy(t_hbm.at[src_v], rows_v, sem).wait()
        pltpu.sync_copy(rows_v, acc.at[dst_v], add=True)

    plsc.subcore_barrier()

    @pl.loop(0, _STR // _K)
    def _out(j):
        row = s * _STR + j * _K
        pltpu.sync_copy(acc.at[pl.ds(row, _K)], rows_v)
        pltpu.sync_copy(rows_v, out_hbm.at[pl.ds(c * _NP + row, _K)])


def _build_deg_count(interpret=False):
    return pl.kernel(
        _deg_count_body,
        out_type=jax.ShapeDtypeStruct((_NC * _NP, _D), jnp.float32),
        mesh=_mesh,
        scratch_types=[
            pltpu.VMEM_SHARED((_NP, _D), jnp.float32),
            pltpu.VMEM((_K,), jnp.int32),
            pltpu.VMEM((_K, _D), jnp.float32),
        ],
        interpret=interpret,
    )


def _build_edge_scatter(interpret=False):
    return pl.kernel(
        _edge_scatter_body,
        out_type=jax.ShapeDtypeStruct((_NC * _NP, _D), jnp.float32),
        mesh=_mesh,
        scratch_types=[
            pltpu.VMEM_SHARED((_NP, _D), jnp.float32),
            pltpu.VMEM((_K,), jnp.int32),
            pltpu.VMEM((_K,), jnp.int32),
            pltpu.VMEM((_K, _D), jnp.float32),
            pltpu.SemaphoreType.DMA,
        ],
        interpret=interpret,
    )


_deg_count = _build_deg_count()
_edge_scatter = _build_edge_scatter()


def _dinv_block(degp_blk):
    deg = degp_blk[0, :, 0] + degp_blk[1, :, 0] + 1.0
    return lax.rsqrt(deg)[:, None]


def _stage1_body(x_ref, w_ref, degp_ref, t_ref):
    dinv = _dinv_block(degp_ref[...])
    t_ref[...] = jnp.dot(x_ref[...], w_ref[...],
                         preferred_element_type=jnp.float32) * dinv


def _stage_mid_body(acc_ref, t_ref, degp_ref, b_ref, w_ref, out_ref):
    dinv = _dinv_block(degp_ref[...])
    prev = (acc_ref[0] + acc_ref[1] + t_ref[...]) * dinv + b_ref[...]
    z = jnp.maximum(prev, 0.0)
    out_ref[...] = jnp.dot(z, w_ref[...],
                           preferred_element_type=jnp.float32) * dinv


def _final_body(acc_ref, t_ref, degp_ref, b_ref, batch_ref, wl_ref, bl_ref,
                out_ref, sums, cnt):
    i = pl.program_id(0)

    @pl.when(i == 0)
    def _():
        sums[...] = jnp.zeros_like(sums)
        cnt[...] = jnp.zeros_like(cnt)

    dinv = _dinv_block(degp_ref[...])
    h3 = (acc_ref[0] + acc_ref[1] + t_ref[...]) * dinv + b_ref[...]
    bt = batch_ref[0, 0, :]
    gid = lax.broadcasted_iota(jnp.int32, (_G, 1), 0)
    oh = (bt[None, :] == gid).astype(jnp.float32)
    sums[...] += jnp.dot(oh, h3, preferred_element_type=jnp.float32)
    cnt[...] += jnp.broadcast_to(jnp.sum(oh, axis=1, keepdims=True),
                                 (_G, _D))

    @pl.when(i == _NB - 1)
    def _():
        pooled = sums[...] / jnp.clip(cnt[...], 1.0)
        out_ref[...] = jnp.dot(pooled, wl_ref[...],
                               preferred_element_type=jnp.float32) + bl_ref[...]


def _rows_spec():
    return pl.BlockSpec((_BR, _D), lambda i: (i, 0))


def _acc_spec():
    return pl.BlockSpec((_NC, _BR, _D), lambda i: (0, i, 0))


def _degp_spec():
    return pl.BlockSpec((_NC, _BR, _D), lambda i: (0, i, 0))


def _full_spec(shape):
    return pl.BlockSpec(shape, lambda i: tuple(0 for _ in shape))


def _stage1(x_p, w1, degp):
    return pl.pallas_call(
        _stage1_body,
        grid=(_NB,),
        in_specs=[_rows_spec(), _full_spec((_D, _D)), _degp_spec()],
        out_specs=_rows_spec(),
        out_shape=jax.ShapeDtypeStruct((_NP, _D), jnp.float32),
    )(x_p, w1, degp)


def _stage_mid(acc, t_prev, degp, b, w):
    return pl.pallas_call(
        _stage_mid_body,
        grid=(_NB,),
        in_specs=[_acc_spec(), _rows_spec(), _degp_spec(),
                  _full_spec((1, _D)), _full_spec((_D, _D))],
        out_specs=_rows_spec(),
        out_shape=jax.ShapeDtypeStruct((_NP, _D), jnp.float32),
    )(acc, t_prev, degp, b, w)


def _final(acc, t3, degp, b3, batch3d, wl, bl):
    return pl.pallas_call(
        _final_body,
        grid=(_NB,),
        in_specs=[_acc_spec(), _rows_spec(), _degp_spec(),
                  _full_spec((1, _D)),
                  pl.BlockSpec((1, 1, _BR), lambda i: (i, 0, 0)),
                  _full_spec((_D, _C)), _full_spec((1, _C))],
        out_specs=_full_spec((_G, _C)),
        out_shape=jax.ShapeDtypeStruct((_G, _C), jnp.float32),
        scratch_shapes=[pltpu.VMEM((_G, _D), jnp.float32),
                        pltpu.VMEM((_G, _D), jnp.float32)],
    )(acc, t3, degp, b3, batch3d, wl, bl)


@jax.jit
def kernel(x, edge_index, batch, W1, b1, W2, b2, W3, b3, Wl, bl):
    src = edge_index[0]
    dst = edge_index[1]
    x_p = jnp.pad(x, ((0, _NP - _N), (0, 0)))
    batch3d = jnp.pad(batch, (0, _NP - _N), constant_values=-1).reshape(
        _NB, 1, _BR)
    zrows = jnp.zeros((_K, _D), jnp.float32)
    orows = jnp.ones((_K, _D), jnp.float32)

    degp = _deg_count(dst, orows, zrows).reshape(_NC, _NP, _D)

    t1 = _stage1(x_p, W1, degp)
    acc1 = _edge_scatter(t1, src, dst, zrows).reshape(_NC, _NP, _D)
    t2 = _stage_mid(acc1, t1, degp, b1.reshape(1, _D), W2)
    acc2 = _edge_scatter(t2, src, dst, zrows).reshape(_NC, _NP, _D)
    t3 = _stage_mid(acc2, t2, degp, b2.reshape(1, _D), W3)
    acc3 = _edge_scatter(t3, src, dst, zrows).reshape(_NC, _NP, _D)

    return _final(acc3, t3, degp, b3.reshape(1, _D), batch3d,
                  Wl, bl.reshape(1, _C))


# speedup vs baseline: 22.1378x; 2.1362x over previous
import functools

import jax
import jax.numpy as jnp
from jax import lax
from jax.experimental import pallas as pl
from jax.experimental.pallas import tpu as pltpu
from jax.experimental.pallas import tpu_sc as plsc

_N = 10000
_E = 320000
_D = 128
_C = 19
_G = 64

_NC = 2
_NS = 16
_NW = _NC * _NS
_NP = 10240
_STR = _NP // _NS
_K = 80
_EPT = _E // _NW
_CH = _EPT // _K
_NSL = 4
_SG = (_CH - 1) // _NSL

_BR = 1024
_NB = _NP // _BR

_mesh = plsc.VectorSubcoreMesh(core_axis_name="c", subcore_axis_name="s",
                               num_cores=_NC, num_subcores=_NS)


def _deg_count_body(dst_hbm, ones_hbm, zrows_hbm, out_hbm,
                    acc, d0, d1, d2, d3, rows_v,
                    id0, id1, id2, id3, s0, s1, s2, s3):
    dslot = (d0, d1, d2, d3)
    idsem = (id0, id1, id2, id3)
    ssem = (s0, s1, s2, s3)
    c = lax.axis_index("c")
    s = lax.axis_index("s")
    base = (c * _NS + s) * _EPT

    def idx_load(chunk, q):
        pltpu.async_copy(dst_hbm.at[pl.ds(base + chunk * _K, _K)], dslot[q],
                         idsem[q])

    def idx_wait(chunk, q):
        pltpu.make_async_copy(dst_hbm.at[pl.ds(base + chunk * _K, _K)],
                              dslot[q], idsem[q]).wait()

    for q in range(_NSL):
        idx_load(q, q)

    pltpu.sync_copy(zrows_hbm, rows_v)

    @pl.loop(0, _STR // _K)
    def _zero(j):
        pltpu.sync_copy(rows_v, acc.at[pl.ds(s * _STR + j * _K, _K)])

    plsc.subcore_barrier()

    pltpu.sync_copy(ones_hbm, rows_v)

    @pl.loop(0, _SG)
    def _count(gg):
        for q in range(_NSL):
            i = gg * _NSL + q
            idx_wait(i, q)
            pltpu.async_copy(rows_v, acc.at[dslot[q]], ssem[q], add=True)

            @pl.when(i + _NSL < _CH)
            def _():
                pltpu.make_async_copy(rows_v, acc.at[dslot[q]],
                                      ssem[q]).wait()
                idx_load(i + _NSL, q)

    last = _CH - 1
    idx_wait(last, 0)
    pltpu.async_copy(rows_v, acc.at[dslot[0]], ssem[0], add=True)
    for q in range(_NSL):
        pltpu.make_async_copy(rows_v, acc.at[dslot[q]], ssem[q]).wait()

    plsc.subcore_barrier()

    @pl.loop(0, _STR // _K)
    def _out(j):
        row = s * _STR + j * _K
        pltpu.sync_copy(acc.at[pl.ds(row, _K)], rows_v)
        pltpu.sync_copy(rows_v, out_hbm.at[pl.ds(c * _NP + row, _K)])


def _edge_scatter_body(t_hbm, src_hbm, dst_hbm, zrows_hbm, out_hbm,
                       acc, sv0, sv1, sv2, sv3, dv0, dv1, dv2, dv3,
                       r0, r1, r2, r3,
                       gs0, gs1, gs2, gs3, ss0, ss1, ss2, ss3,
                       is0, is1, is2, is3, id0, id1, id2, id3):
    sslot = (sv0, sv1, sv2, sv3)
    dslot = (dv0, dv1, dv2, dv3)
    rows = (r0, r1, r2, r3)
    gsem = (gs0, gs1, gs2, gs3)
    ssem = (ss0, ss1, ss2, ss3)
    issem = (is0, is1, is2, is3)
    idsem = (id0, id1, id2, id3)
    c = lax.axis_index("c")
    s = lax.axis_index("s")
    base = (c * _NS + s) * _EPT

    def idx_load(chunk, q):
        pltpu.async_copy(src_hbm.at[pl.ds(base + chunk * _K, _K)], sslot[q],
                         issem[q])
        pltpu.async_copy(dst_hbm.at[pl.ds(base + chunk * _K, _K)], dslot[q],
                         idsem[q])

    def idx_wait(sems, slots, chunk, q):
        pltpu.make_async_copy(src_hbm.at[pl.ds(base + chunk * _K, _K)],
                              slots[q], sems[q]).wait()

    for q in range(_NSL):
        idx_load(q, q)

    pltpu.sync_copy(zrows_hbm, rows[0])

    @pl.loop(0, _STR // _K)
    def _zero(j):
        pltpu.sync_copy(rows[0], acc.at[pl.ds(s * _STR + j * _K, _K)])

    plsc.subcore_barrier()

    for q in range(_NSL):
        idx_wait(issem, sslot, q, q)
        pltpu.async_copy(t_hbm.at[sslot[q]], rows[q], gsem[q])

    @pl.loop(0, _SG)
    def _edges(gg):
        for q in range(_NSL):
            i = gg * _NSL + q
            pltpu.make_async_copy(t_hbm.at[sslot[q]], rows[q],
                                  gsem[q]).wait()
            idx_wait(idsem, dslot, i, q)
            pltpu.async_copy(rows[q], acc.at[dslot[q]], ssem[q], add=True)
            pltpu.make_async_copy(rows[q], acc.at[dslot[q]],
                                  ssem[q]).wait()

            @pl.when(i + _NSL < _CH)
            def _():
                idx_load(i + _NSL, q)
                idx_wait(issem, sslot, i + _NSL, q)
                pltpu.async_copy(t_hbm.at[sslot[q]], rows[q], gsem[q])

    last = _CH - 1
    pltpu.make_async_copy(t_hbm.at[sslot[0]], rows[0], gsem[0]).wait()
    idx_wait(idsem, dslot, last, 0)
    pltpu.sync_copy(rows[0], acc.at[dslot[0]], add=True)

    plsc.subcore_barrier()

    @pl.loop(0, _STR // _K)
    def _out(j):
        row = s * _STR + j * _K
        pltpu.sync_copy(acc.at[pl.ds(row, _K)], rows[0])
        pltpu.sync_copy(rows[0], out_hbm.at[pl.ds(c * _NP + row, _K)])


def _build_deg_count(interpret=False):
    return pl.kernel(
        _deg_count_body,
        out_type=jax.ShapeDtypeStruct((_NC * _NP, _D), jnp.float32),
        mesh=_mesh,
        scratch_types=(
            [pltpu.VMEM_SHARED((_NP, _D), jnp.float32)]
            + [pltpu.VMEM((_K,), jnp.int32)] * _NSL
            + [pltpu.VMEM((_K, _D), jnp.float32)]
            + [pltpu.SemaphoreType.DMA] * (2 * _NSL)),
        interpret=interpret,
    )


def _build_edge_scatter(interpret=False):
    return pl.kernel(
        _edge_scatter_body,
        out_type=jax.ShapeDtypeStruct((_NC * _NP, _D), jnp.float32),
        mesh=_mesh,
        scratch_types=(
            [pltpu.VMEM_SHARED((_NP, _D), jnp.float32)]
            + [pltpu.VMEM((_K,), jnp.int32)] * (2 * _NSL)
            + [pltpu.VMEM((_K, _D), jnp.float32)] * _NSL
            + [pltpu.SemaphoreType.DMA] * (4 * _NSL)),
        interpret=interpret,
    )


_deg_count = _build_deg_count()
_edge_scatter = _build_edge_scatter()


def _dinv_block(degp_blk):
    deg = degp_blk[0, :, 0] + degp_blk[1, :, 0] + 1.0
    return lax.rsqrt(deg)[:, None]


def _stage1_body(x_ref, w_ref, degp_ref, t_ref):
    dinv = _dinv_block(degp_ref[...])
    t_ref[...] = jnp.dot(x_ref[...], w_ref[...],
                         preferred_element_type=jnp.float32) * dinv


def _stage_mid_body(acc_ref, t_ref, degp_ref, b_ref, w_ref, out_ref):
    dinv = _dinv_block(degp_ref[...])
    prev = (acc_ref[0] + acc_ref[1] + t_ref[...]) * dinv + b_ref[...]
    z = jnp.maximum(prev, 0.0)
    out_ref[...] = jnp.dot(z, w_ref[...],
                           preferred_element_type=jnp.float32) * dinv


def _final_body(acc_ref, t_ref, degp_ref, b_ref, batch_ref, wl_ref, bl_ref,
                out_ref, sums, cnt):
    i = pl.program_id(0)

    @pl.when(i == 0)
    def _():
        sums[...] = jnp.zeros_like(sums)
        cnt[...] = jnp.zeros_like(cnt)

    dinv = _dinv_block(degp_ref[...])
    h3 = (acc_ref[0] + acc_ref[1] + t_ref[...]) * dinv + b_ref[...]
    bt = batch_ref[0, 0, :]
    gid = lax.broadcasted_iota(jnp.int32, (_G, 1), 0)
    oh = (bt[None, :] == gid).astype(jnp.float32)
    sums[...] += jnp.dot(oh, h3, preferred_element_type=jnp.float32)
    cnt[...] += jnp.broadcast_to(jnp.sum(oh, axis=1, keepdims=True),
                                 (_G, _D))

    @pl.when(i == _NB - 1)
    def _():
        pooled = sums[...] / jnp.clip(cnt[...], 1.0)
        out_ref[...] = jnp.dot(pooled, wl_ref[...],
                               preferred_element_type=jnp.float32) + bl_ref[...]


def _rows_spec():
    return pl.BlockSpec((_BR, _D), lambda i: (i, 0))


def _acc_spec():
    return pl.BlockSpec((_NC, _BR, _D), lambda i: (0, i, 0))


def _degp_spec():
    return pl.BlockSpec((_NC, _BR, _D), lambda i: (0, i, 0))


def _full_spec(shape):
    return pl.BlockSpec(shape, lambda i: tuple(0 for _ in shape))


def _stage1(x_p, w1, degp):
    return pl.pallas_call(
        _stage1_body,
        grid=(_NB,),
        in_specs=[_rows_spec(), _full_spec((_D, _D)), _degp_spec()],
        out_specs=_rows_spec(),
        out_shape=jax.ShapeDtypeStruct((_NP, _D), jnp.float32),
    )(x_p, w1, degp)


def _stage_mid(acc, t_prev, degp, b, w):
    return pl.pallas_call(
        _stage_mid_body,
        grid=(_NB,),
        in_specs=[_acc_spec(), _rows_spec(), _degp_spec(),
                  _full_spec((1, _D)), _full_spec((_D, _D))],
        out_specs=_rows_spec(),
        out_shape=jax.ShapeDtypeStruct((_NP, _D), jnp.float32),
    )(acc, t_prev, degp, b, w)


def _final(acc, t3, degp, b3, batch3d, wl, bl):
    return pl.pallas_call(
        _final_body,
        grid=(_NB,),
        in_specs=[_acc_spec(), _rows_spec(), _degp_spec(),
                  _full_spec((1, _D)),
                  pl.BlockSpec((1, 1, _BR), lambda i: (i, 0, 0)),
                  _full_spec((_D, _C)), _full_spec((1, _C))],
        out_specs=_full_spec((_G, _C)),
        out_shape=jax.ShapeDtypeStruct((_G, _C), jnp.float32),
        scratch_shapes=[pltpu.VMEM((_G, _D), jnp.float32),
                        pltpu.VMEM((_G, _D), jnp.float32)],
    )(acc, t3, degp, b3, batch3d, wl, bl)


@jax.jit
def kernel(x, edge_index, batch, W1, b1, W2, b2, W3, b3, Wl, bl):
    src = edge_index[0]
    dst = edge_index[1]
    x_p = jnp.pad(x, ((0, _NP - _N), (0, 0)))
    batch3d = jnp.pad(batch, (0, _NP - _N), constant_values=-1).reshape(
        _NB, 1, _BR)
    zrows = jnp.zeros((_K, _D), jnp.float32)
    orows = jnp.ones((_K, _D), jnp.float32)

    degp = _deg_count(dst, orows, zrows).reshape(_NC, _NP, _D)

    t1 = _stage1(x_p, W1, degp)
    acc1 = _edge_scatter(t1, src, dst, zrows).reshape(_NC, _NP, _D)
    t2 = _stage_mid(acc1, t1, degp, b1.reshape(1, _D), W2)
    acc2 = _edge_scatter(t2, src, dst, zrows).reshape(_NC, _NP, _D)
    t3 = _stage_mid(acc2, t2, degp, b2.reshape(1, _D), W3)
    acc3 = _edge_scatter(t3, src, dst, zrows).reshape(_NC, _NP, _D)

    return _final(acc3, t3, degp, b3.reshape(1, _D), batch3d,
                  Wl, bl.reshape(1, _C))
